# Initial kernel scaffold; baseline (speedup 1.0000x reference)
import jax
import jax.numpy as jnp
from jax.experimental import pallas as pl


def kernel(x, edge_index, W_self1, W_neigh1, b1, W_self2, W_neigh2, b2):
    raise NotImplementedError("write your pallas kernel here")


# speedup vs baseline: 5.2513x; 5.2513x over previous
import functools

import jax
import jax.numpy as jnp
from jax import lax
from jax.experimental import pallas as pl
from jax.experimental.pallas import tpu as pltpu
from jax.experimental.pallas import tpu_sc as plsc

N = 10000
E = 320000
D = 128

NC = 2
NS = 16
NW = NC * NS
EPW = E // NW
CH = 80
NCHUNK = EPW // CH
RPT = 624
TAIL = N - NS * RPT

_mesh = plsc.VectorSubcoreMesh(
    core_axis_name="c", subcore_axis_name="s", num_cores=NC, num_subcores=NS
)

_f32 = jnp.float32


def _worker_id():
    return lax.axis_index("c") * NS + lax.axis_index("s")


_GATHER_DNUMS = lax.GatherDimensionNumbers(
    offset_dims=(), collapsed_slice_dims=(0,), start_index_map=(0,))


def _lane_perm(v, idx):
    return lax.gather(v, idx[:, None], _GATHER_DNUMS, slice_sizes=(1,),
                      mode=lax.GatherScatterMode.PROMISE_IN_BOUNDS)


def _make_segsum(with_deg):
    out_type = jax.ShapeDtypeStruct((NC, N, D), _f32)
    scratch = [
        pltpu.VMEM((NCHUNK, CH), jnp.int32),
        pltpu.VMEM((NCHUNK, CH), jnp.int32),
        pltpu.VMEM((CH, D), _f32),
        pltpu.VMEM_SHARED((N, D), _f32),
    ]
    if with_deg:
        out_type = (out_type, jax.ShapeDtypeStruct((NC * N,), _f32))
        scratch += [
            pltpu.VMEM((CH,), _f32),
            pltpu.VMEM_SHARED((N,), _f32),
            pltpu.VMEM((RPT + TAIL,), _f32),
        ]

    @functools.partial(
        pl.kernel, out_type=out_type, mesh=_mesh,
        scratch_types=tuple(scratch),
    )
    def segsum(*refs):
        if with_deg:
            (feat, src2d, dst2d, znd, acc_out, deg_out,
             idx_s, idx_d, rows, acc_sh, ones_v, deg_sh, deg_buf) = refs
        else:
            (feat, src2d, dst2d, znd, acc_out,
             idx_s, idx_d, rows, acc_sh) = refs

        c = lax.axis_index("c")
        s = lax.axis_index("s")
        w = c * NS + s

        pltpu.sync_copy(znd.at[pl.ds(s * RPT, RPT)],
                        acc_sh.at[pl.ds(s * RPT, RPT)])

        @pl.when(s == NS - 1)
        def _():
            pltpu.sync_copy(znd.at[pl.ds(NS * RPT, TAIL)],
                            acc_sh.at[pl.ds(NS * RPT, TAIL)])

        if with_deg:
            for j in range((RPT + TAIL) // 16):
                deg_buf[pl.ds(j * 16, 16)] = jnp.zeros((16,), _f32)
            pltpu.sync_copy(deg_buf.at[pl.ds(0, RPT)],
                            deg_sh.at[pl.ds(s * RPT, RPT)])

            @pl.when(s == NS - 1)
            def _():
                pltpu.sync_copy(deg_buf.at[pl.ds(0, TAIL)],
                                deg_sh.at[pl.ds(NS * RPT, TAIL)])

            for j in range(CH // 16):
                ones_v[pl.ds(j * 16, 16)] = jnp.ones((16,), _f32)
        plsc.subcore_barrier()

        pltpu.sync_copy(src2d.at[w], idx_s)
        pltpu.sync_copy(dst2d.at[w], idx_d)

        def step(i, carry):
            pltpu.sync_copy(feat.at[idx_s.at[i]], rows)
            pltpu.sync_copy(rows, acc_sh.at[idx_d.at[i]], add=True)
            if with_deg:
                pltpu.sync_copy(ones_v, deg_sh.at[idx_d.at[i]], add=True)
            return carry

        lax.fori_loop(0, NCHUNK, step, 0)
        plsc.subcore_barrier()

        pltpu.sync_copy(acc_sh.at[pl.ds(s * RPT, RPT)],
                        acc_out.at[c, pl.ds(s * RPT, RPT)])

        @pl.when(s == NS - 1)
        def _():
            pltpu.sync_copy(acc_sh.at[pl.ds(NS * RPT, TAIL)],
                            acc_out.at[c, pl.ds(NS * RPT, TAIL)])

        if with_deg:
            pltpu.sync_copy(deg_sh.at[pl.ds(s * RPT, RPT)],
                            deg_buf.at[pl.ds(0, RPT)])
            pltpu.sync_copy(deg_buf.at[pl.ds(0, RPT)],
                            deg_out.at[pl.ds(c * N + s * RPT, RPT)])

            @pl.when(s == NS - 1)
            def _():
                pltpu.sync_copy(deg_sh.at[pl.ds(NS * RPT, TAIL)],
                                deg_buf.at[pl.ds(0, TAIL)])
                pltpu.sync_copy(deg_buf.at[pl.ds(0, TAIL)],
                                deg_out.at[pl.ds(c * N + NS * RPT, TAIL)])

    return segsum


_segsum_deg = _make_segsum(True)
_segsum = _make_segsum(False)


@functools.partial(
    pl.kernel,
    out_type=jax.ShapeDtypeStruct((E,), _f32),
    mesh=_mesh,
    scratch_types=(
        pltpu.VMEM((NCHUNK, CH), jnp.int32),
        pltpu.VMEM((NCHUNK, CH), jnp.int32),
        pltpu.VMEM((CH, D), _f32),
        pltpu.VMEM((CH, D), _f32),
        pltpu.VMEM((CH,), _f32),
        pltpu.VMEM((256,), _f32),
    ),
)
def _edge_dot(feat, src2d, dst2d, out, idx_s, idx_d, rows_s, rows_d, out_v,
              tmp):
    w = _worker_id()
    pltpu.sync_copy(src2d.at[w], idx_s)
    pltpu.sync_copy(dst2d.at[w], idx_d)

    def step(i, carry):
        pltpu.sync_copy(feat.at[idx_s.at[i]], rows_s)
        pltpu.sync_copy(feat.at[idx_d.at[i]], rows_d)

        def group16(q, carry2):
            lanes = lax.iota(jnp.int32, 16)
            acc = jnp.zeros((16,), _f32)
            for m in range(16):
                e = q * 16 + m
                v = rows_s[e, pl.ds(0, 16)] * rows_d[e, pl.ds(0, 16)]
                for j in range(1, D // 16):
                    v = v + (rows_s[e, pl.ds(j * 16, 16)]
                             * rows_d[e, pl.ds(j * 16, 16)])
                for k in (1, 2, 4, 8):
                    v = v + _lane_perm(v, jnp.remainder(lanes + k, 16))
                acc = jnp.where(lanes == m, v, acc)
            out_v[pl.ds(q * 16, 16)] = acc
            return carry2

        lax.fori_loop(0, CH // 16, group16, 0)
        pltpu.sync_copy(out_v, out.at[pl.ds((w * NCHUNK + i) * CH, CH)])
        return carry

    lax.fori_loop(0, NCHUNK, step, 0)


RB = 1000


def _tc_layer_body(relu, x_ref, a0, a1, d0, d1, ws, wn, b, o_ref):
    deg = jnp.maximum(d0[...] + d1[...], 1.0)
    neigh = (a0[...] + a1[...]) / deg
    h = (jnp.dot(x_ref[...], ws[...], preferred_element_type=_f32)
         + jnp.dot(neigh, wn[...], preferred_element_type=_f32)
         + b[...])
    if relu:
        h = jnp.maximum(h, 0.0)
    o_ref[...] = h


def _tc_layer(x, acc, deg, w_self, w_neigh, b, relu):
    row_spec = pl.BlockSpec((RB, D), lambda i: (i, 0))
    col_spec = pl.BlockSpec((RB, 1), lambda i: (i, 0))
    w_spec = pl.BlockSpec((D, D), lambda i: (0, 0))
    b_spec = pl.BlockSpec((1, D), lambda i: (0, 0))
    return pl.pallas_call(
        functools.partial(_tc_layer_body, relu),
        grid=(N // RB,),
        in_specs=[row_spec, row_spec, row_spec, col_spec, col_spec,
                  w_spec, w_spec, b_spec],
        out_specs=row_spec,
        out_shape=jax.ShapeDtypeStruct((N, D), _f32),
    )(x, acc[0], acc[1], deg[0].reshape(N, 1), deg[1].reshape(N, 1),
      w_self, w_neigh, b.reshape(1, D))


def kernel(x, edge_index, W_self1, W_neigh1, b1, W_self2, W_neigh2, b2):
    src2d = edge_index[0].reshape(NW, NCHUNK, CH)
    dst2d = edge_index[1].reshape(NW, NCHUNK, CH)
    znd = jnp.zeros((N, D), _f32)

    accx, degp = _segsum_deg(x, src2d, dst2d, znd)
    degp = degp.reshape(NC, N)
    h = _tc_layer(x, accx, degp, W_self1, W_neigh1, b1, relu=True)
    acch = _segsum(h, src2d, dst2d, znd)
    h2 = _tc_layer(h, acch, degp, W_self2, W_neigh2, b2, relu=False)
    score = _edge_dot(h2, src2d, dst2d)
    return score.reshape(E, 1)


# speedup vs baseline: 7.8726x; 1.4992x over previous
import functools

import jax
import jax.numpy as jnp
from jax import lax
from jax.experimental import pallas as pl
from jax.experimental.pallas import tpu as pltpu
from jax.experimental.pallas import tpu_sc as plsc

N = 10000
E = 320000
D = 128

NC = 2
NS = 16
NW = NC * NS
EPW = E // NW
CH = 80
NCHUNK = EPW // CH
BLKR = 25
NBLK = NCHUNK // BLKR
RPT = 624
TAIL = N - NS * RPT

_mesh = plsc.VectorSubcoreMesh(
    core_axis_name="c", subcore_axis_name="s", num_cores=NC, num_subcores=NS
)

_f32 = jnp.float32


def _worker_id():
    return lax.axis_index("c") * NS + lax.axis_index("s")


_GATHER_DNUMS = lax.GatherDimensionNumbers(
    offset_dims=(), collapsed_slice_dims=(0,), start_index_map=(0,))


def _lane_perm(v, idx):
    return lax.gather(v, idx[:, None], _GATHER_DNUMS, slice_sizes=(1,),
                      mode=lax.GatherScatterMode.PROMISE_IN_BOUNDS)


def _make_segsum(with_deg):
    out_type = jax.ShapeDtypeStruct((NC, N, D), _f32)
    scratch = [
        pltpu.VMEM((BLKR, CH), jnp.int32),
        pltpu.VMEM((BLKR, CH), jnp.int32),
        pltpu.VMEM((CH, D), _f32),
        pltpu.VMEM((CH, D), _f32),
        pltpu.SemaphoreType.DMA,
        pltpu.SemaphoreType.DMA,
        pltpu.VMEM_SHARED((N, D), _f32),
    ]
    if with_deg:
        out_type = (out_type, jax.ShapeDtypeStruct((NC * N,), _f32))
        scratch += [
            pltpu.VMEM((CH,), _f32),
            pltpu.VMEM_SHARED((N,), _f32),
            pltpu.VMEM((RPT + TAIL,), _f32),
        ]

    @functools.partial(
        pl.kernel, out_type=out_type, mesh=_mesh,
        scratch_types=tuple(scratch),
    )
    def segsum(*refs):
        if with_deg:
            (feat, src2d, dst2d, znd, acc_out, deg_out,
             idx_s, idx_d, rows0, rows1, sem0, sem1, acc_sh,
             ones_v, deg_sh, deg_buf) = refs
        else:
            (feat, src2d, dst2d, znd, acc_out,
             idx_s, idx_d, rows0, rows1, sem0, sem1, acc_sh) = refs
        rows = (rows0, rows1)
        sems = (sem0, sem1)

        c = lax.axis_index("c")
        s = lax.axis_index("s")
        w = c * NS + s

        pltpu.sync_copy(znd.at[pl.ds(s * RPT, RPT)],
                        acc_sh.at[pl.ds(s * RPT, RPT)])

        @pl.when(s == NS - 1)
        def _():
            pltpu.sync_copy(znd.at[pl.ds(NS * RPT, TAIL)],
                            acc_sh.at[pl.ds(NS * RPT, TAIL)])

        if with_deg:
            for j in range((RPT + TAIL) // 16):
                deg_buf[pl.ds(j * 16, 16)] = jnp.zeros((16,), _f32)
            pltpu.sync_copy(deg_buf.at[pl.ds(0, RPT)],
                            deg_sh.at[pl.ds(s * RPT, RPT)])

            @pl.when(s == NS - 1)
            def _():
                pltpu.sync_copy(deg_buf.at[pl.ds(0, TAIL)],
                                deg_sh.at[pl.ds(NS * RPT, TAIL)])

            for j in range(CH // 16):
                ones_v[pl.ds(j * 16, 16)] = jnp.ones((16,), _f32)
        plsc.subcore_barrier()

        for blk in range(NBLK):
            pltpu.sync_copy(src2d.at[w, blk], idx_s)
            pltpu.sync_copy(dst2d.at[w, blk], idx_d)

            for b in range(2):
                pltpu.async_copy(feat.at[idx_s.at[b]], rows[b], sems[b])

            def consume(c, b):
                pltpu.make_async_copy(feat.at[idx_s.at[c]], rows[b],
                                      sems[b]).wait()
                pltpu.sync_copy(rows[b], acc_sh.at[idx_d.at[c]], add=True)
                if with_deg:
                    pltpu.sync_copy(ones_v, deg_sh.at[idx_d.at[c]],
                                    add=True)

                @pl.when(c + 2 < BLKR)
                def _():
                    pltpu.async_copy(feat.at[idx_s.at[c + 2]], rows[b],
                                     sems[b])

            def pair(i2, carry):
                for b in range(2):
                    consume(i2 * 2 + b, b)
                return carry

            lax.fori_loop(0, (BLKR - 1) // 2, pair, 0)
            consume(BLKR - 1, (BLKR - 1) % 2)
        plsc.subcore_barrier()

        pltpu.sync_copy(acc_sh.at[pl.ds(s * RPT, RPT)],
                        acc_out.at[c, pl.ds(s * RPT, RPT)])

        @pl.when(s == NS - 1)
        def _():
            pltpu.sync_copy(acc_sh.at[pl.ds(NS * RPT, TAIL)],
                            acc_out.at[c, pl.ds(NS * RPT, TAIL)])

        if with_deg:
            pltpu.sync_copy(deg_sh.at[pl.ds(s * RPT, RPT)],
                            deg_buf.at[pl.ds(0, RPT)])
            pltpu.sync_copy(deg_buf.at[pl.ds(0, RPT)],
                            deg_out.at[pl.ds(c * N + s * RPT, RPT)])

            @pl.when(s == NS - 1)
            def _():
                pltpu.sync_copy(deg_sh.at[pl.ds(NS * RPT, TAIL)],
                                deg_buf.at[pl.ds(0, TAIL)])
                pltpu.sync_copy(deg_buf.at[pl.ds(0, TAIL)],
                                deg_out.at[pl.ds(c * N + NS * RPT, TAIL)])

    return segsum


_segsum_deg = _make_segsum(True)
_segsum = _make_segsum(False)


@functools.partial(
    pl.kernel,
    out_type=jax.ShapeDtypeStruct((E,), _f32),
    mesh=_mesh,
    scratch_types=(
        pltpu.VMEM((BLKR, CH), jnp.int32),
        pltpu.VMEM((BLKR, CH), jnp.int32),
        pltpu.VMEM((CH, D), _f32),
        pltpu.VMEM((CH, D), _f32),
        pltpu.VMEM((CH, D), _f32),
        pltpu.VMEM((CH, D), _f32),
        pltpu.SemaphoreType.DMA,
        pltpu.SemaphoreType.DMA,
        pltpu.SemaphoreType.DMA,
        pltpu.SemaphoreType.DMA,
        pltpu.VMEM((CH,), _f32),
    ),
)
def _edge_dot(feat, src2d, dst2d, out, idx_s, idx_d, rs0, rs1, rd0, rd1,
              ss0, ss1, sd0, sd1, out_v):
    rows_s = (rs0, rs1)
    rows_d = (rd0, rd1)
    sem_s = (ss0, ss1)
    sem_d = (sd0, sd1)
    w = _worker_id()

    def block(blk, carry0):
        pltpu.sync_copy(src2d.at[w, blk], idx_s)
        pltpu.sync_copy(dst2d.at[w, blk], idx_d)

        for b in range(2):
            pltpu.async_copy(feat.at[idx_s.at[b]], rows_s[b], sem_s[b])
            pltpu.async_copy(feat.at[idx_d.at[b]], rows_d[b], sem_d[b])

        def consume(c, b):
            pltpu.make_async_copy(feat.at[idx_s.at[c]], rows_s[b],
                                  sem_s[b]).wait()
            pltpu.make_async_copy(feat.at[idx_d.at[c]], rows_d[b],
                                  sem_d[b]).wait()

            def group16(q, carry2):
                lanes = lax.iota(jnp.int32, 16)
                acc = jnp.zeros((16,), _f32)
                for m in range(16):
                    e = q * 16 + m
                    v = (rows_s[b][e, pl.ds(0, 16)]
                         * rows_d[b][e, pl.ds(0, 16)])
                    for j in range(1, D // 16):
                        v = v + (rows_s[b][e, pl.ds(j * 16, 16)]
                                 * rows_d[b][e, pl.ds(j * 16, 16)])
                    for k in (1, 2, 4, 8):
                        v = v + _lane_perm(v, jnp.remainder(lanes + k, 16))
                    acc = jnp.where(lanes == m, v, acc)
                out_v[pl.ds(q * 16, 16)] = acc
                return carry2

            lax.fori_loop(0, CH // 16, group16, 0)
            pltpu.sync_copy(
                out_v,
                out.at[pl.ds((w * NCHUNK + blk * BLKR + c) * CH, CH)])

            @pl.when(c + 2 < BLKR)
            def _():
                pltpu.async_copy(feat.at[idx_s.at[c + 2]], rows_s[b],
                                 sem_s[b])
                pltpu.async_copy(feat.at[idx_d.at[c + 2]], rows_d[b],
                                 sem_d[b])

        def pairj(j, carry):
            consume(j * 2, 0)

            @pl.when(j * 2 + 1 < BLKR)
            def _():
                consume(j * 2 + 1, 1)

            return carry

        lax.fori_loop(0, (BLKR + 1) // 2, pairj, 0)
        return carry0

    lax.fori_loop(0, NBLK, block, 0)


RB = 1000


def _tc_layer_body(relu, x_ref, a0, a1, d0, d1, ws, wn, b, o_ref):
    deg = jnp.maximum(d0[...] + d1[...], 1.0)
    neigh = (a0[...] + a1[...]) / deg
    h = (jnp.dot(x_ref[...], ws[...], preferred_element_type=_f32)
         + jnp.dot(neigh, wn[...], preferred_element_type=_f32)
         + b[...])
    if relu:
        h = jnp.maximum(h, 0.0)
    o_ref[...] = h


def _tc_layer(x, acc, deg, w_self, w_neigh, b, relu):
    row_spec = pl.BlockSpec((RB, D), lambda i: (i, 0))
    col_spec = pl.BlockSpec((RB, 1), lambda i: (i, 0))
    w_spec = pl.BlockSpec((D, D), lambda i: (0, 0))
    b_spec = pl.BlockSpec((1, D), lambda i: (0, 0))
    return pl.pallas_call(
        functools.partial(_tc_layer_body, relu),
        grid=(N // RB,),
        in_specs=[row_spec, row_spec, row_spec, col_spec, col_spec,
                  w_spec, w_spec, b_spec],
        out_specs=row_spec,
        out_shape=jax.ShapeDtypeStruct((N, D), _f32),
    )(x, acc[0], acc[1], deg[0].reshape(N, 1), deg[1].reshape(N, 1),
      w_self, w_neigh, b.reshape(1, D))


def kernel(x, edge_index, W_self1, W_neigh1, b1, W_self2, W_neigh2, b2):
    src2d = edge_index[0].reshape(NW, NBLK, BLKR, CH)
    dst2d = edge_index[1].reshape(NW, NBLK, BLKR, CH)
    znd = jnp.zeros((N, D), _f32)

    accx, degp = _segsum_deg(x, src2d, dst2d, znd)
    degp = degp.reshape(NC, N)
    h = _tc_layer(x, accx, degp, W_self1, W_neigh1, b1, relu=True)
    acch = _segsum(h, src2d, dst2d, znd)
    h2 = _tc_layer(h, acch, degp, W_self2, W_neigh2, b2, relu=False)
    score = _edge_dot(h2, src2d, dst2d)
    return score.reshape(E, 1)
